# Initial kernel scaffold; baseline (speedup 1.0000x reference)
import jax
import jax.numpy as jnp
from jax.experimental import pallas as pl


def kernel(x, ed_idx, cls, bias):
    raise NotImplementedError("write your pallas kernel here")


# speedup vs baseline: 87.3561x; 87.3561x over previous
import functools

import jax
import jax.numpy as jnp
from jax import lax
from jax.experimental import pallas as pl
from jax.experimental.pallas import tpu as pltpu
from jax.experimental.pallas import tpu_sc as plsc

_LANES = 16


def _sc_segment_sums(x, ed_idx, *, chunk_rows=256):
    n, d = x.shape
    b = ed_idx.shape[0]
    nc, ns = 2, 16
    nw = nc * ns
    assert b % nw == 0 and d % _LANES == 0
    spw = b // nw
    assert spw == _LANES
    dv = d // _LANES

    mesh = plsc.VectorSubcoreMesh(core_axis_name="c", subcore_axis_name="s")

    @functools.partial(
        pl.kernel,
        out_type=jax.ShapeDtypeStruct((b, d), jnp.float32),
        mesh=mesh,
        scratch_types=[
            pltpu.VMEM((b,), jnp.int32),
            pltpu.VMEM((chunk_rows, d), jnp.float32),
            pltpu.VMEM((b // nw, d), jnp.float32),
        ],
    )
    def seg_sum_kernel(x_hbm, ed_hbm, out_hbm, ed_v, chunk_v, rows_v):
        wid = lax.axis_index("s") * nc + lax.axis_index("c")
        pltpu.sync_copy(ed_hbm, ed_v)

        grp = ed_v[pl.ds(wid * spw, spw)]
        pg = ed_v[pl.ds(jnp.maximum(wid - 1, 0) * spw, spw)]
        prev = jnp.where(wid == 0, 0, pg[spw - 1])

        for k in range(spw):
            end = grp[k]
            start = prev if k == 0 else grp[k - 1]
            astart = (start // 8) * 8
            nch = lax.div(end - astart + (chunk_rows - 1), chunk_rows)

            def chunk_body(c, accs, start=start, end=end, astart=astart):
                cbase = astart + c * chunk_rows
                cbase_cl = jnp.minimum(cbase, n - chunk_rows)
                pltpu.sync_copy(x_hbm.at[pl.ds(cbase_cl, chunk_rows)], chunk_v)
                lo = jnp.maximum(start, cbase) - cbase_cl
                hi = jnp.minimum(end, cbase + chunk_rows) - cbase_cl

                def row_body(r, accs):
                    return tuple(
                        accs[t] + chunk_v[r, pl.ds(t * _LANES, _LANES)]
                        for t in range(dv)
                    )

                return lax.fori_loop(lo, hi, row_body, accs)

            zero = jnp.zeros((_LANES,), jnp.float32)
            accs = lax.fori_loop(0, nch, chunk_body, (zero,) * dv)
            for t in range(dv):
                rows_v[k, pl.ds(t * _LANES, _LANES)] = accs[t]
        pltpu.sync_copy(rows_v, out_hbm.at[pl.ds(wid * spw, spw)])

    return seg_sum_kernel(x, ed_idx)


def _tc_tail_kernel(sums_ref, inv_counts_ref, clsT_ref, bias_ref, out_ref):
    ave = sums_ref[...] * inv_counts_ref[...]
    t = ave[:, 0:1]
    inner = jnp.sum(ave * ave, axis=1, keepdims=True) - 2.0 * t * t
    denom = jnp.sqrt(jnp.maximum(jnp.abs(inner), 1e-8))
    cx = ave / denom
    col = lax.broadcasted_iota(jnp.int32, cx.shape, 1)
    cx = jnp.where(col == 0, -cx, cx)
    logits = jnp.dot(cx, clsT_ref[...], preferred_element_type=jnp.float32)
    out_ref[...] = 2.0 + 2.0 * logits + bias_ref[...]


def kernel(x, ed_idx, cls, bias):
    b = ed_idx.shape[0]
    c = cls.shape[0]
    sums = _sc_segment_sums(x, ed_idx)
    starts = jnp.concatenate([jnp.zeros((1,), ed_idx.dtype), ed_idx[:-1]])
    counts = jnp.maximum((ed_idx - starts).astype(jnp.float32), 1.0)
    inv_counts = (1.0 / counts)[:, None]
    out = pl.pallas_call(
        _tc_tail_kernel,
        out_shape=jax.ShapeDtypeStruct((b, c), jnp.float32),
    )(sums, inv_counts, cls.T, bias[None, :])
    return out


# speedup vs baseline: 157.2943x; 1.8006x over previous
import functools

import jax
import jax.numpy as jnp
from jax import lax
from jax.experimental import pallas as pl
from jax.experimental.pallas import tpu as pltpu
from jax.experimental.pallas import tpu_sc as plsc

_LANES = 16


def _sc_segment_sums(x, ed_idx, *, chunk_rows=256):
    n, d = x.shape
    b = ed_idx.shape[0]
    nc, ns = 2, 16
    nw = nc * ns
    assert b % nw == 0 and d % _LANES == 0
    spw = b // nw
    assert spw == _LANES
    dv = d // _LANES

    mesh = plsc.VectorSubcoreMesh(core_axis_name="c", subcore_axis_name="s")

    @functools.partial(
        pl.kernel,
        out_type=jax.ShapeDtypeStruct((b, d), jnp.float32),
        mesh=mesh,
        scratch_types=[
            pltpu.VMEM((b,), jnp.int32),
            pltpu.VMEM((2, chunk_rows, d), jnp.float32),
            pltpu.VMEM((b // nw, d), jnp.float32),
            pltpu.SemaphoreType.DMA((2,)),
        ],
    )
    def seg_sum_kernel(x_hbm, ed_hbm, out_hbm, ed_v, buf, rows_v, sems):
        wid = lax.axis_index("s") * nc + lax.axis_index("c")
        pltpu.sync_copy(ed_hbm, ed_v)

        grp = ed_v[pl.ds(wid * spw, spw)]
        pg = ed_v[pl.ds(jnp.maximum(wid - 1, 0) * spw, spw)]
        prev = jnp.where(wid == 0, 0, pg[spw - 1])
        ends = [grp[k] for k in range(spw)]
        starts = [prev] + ends[:-1]

        zero = jnp.zeros((_LANES,), jnp.float32)
        for k in range(spw):
            for t in range(dv):
                rows_v[k, pl.ds(t * _LANES, _LANES)] = zero

        astart = (starts[0] // 8) * 8
        nch = lax.div(ends[-1] - astart + (chunk_rows - 1), chunk_rows)

        def chunk_dma(c):
            base = jnp.minimum(astart + c * chunk_rows, n - chunk_rows)
            return pltpu.make_async_copy(
                x_hbm.at[pl.ds(base, chunk_rows)], buf.at[c % 2], sems.at[c % 2]
            )

        @pl.when(nch > 0)
        def _():
            chunk_dma(0).start()

        def chunk_body(c, _):
            @pl.when(c + 1 < nch)
            def _():
                chunk_dma(c + 1).start()

            chunk_dma(c).wait()
            cb = astart + c * chunk_rows
            base = jnp.minimum(cb, n - chunk_rows)
            p = c % 2
            for k in range(spw):
                lo = jnp.maximum(starts[k], cb) - base
                hi = jnp.minimum(ends[k], cb + chunk_rows) - base

                @pl.when(lo < hi)
                def _(lo=lo, hi=hi, k=k):
                    def row_body(r, accs):
                        return tuple(
                            accs[t] + buf[p, r, pl.ds(t * _LANES, _LANES)]
                            for t in range(dv)
                        )

                    accs = lax.fori_loop(lo, hi, row_body, (zero,) * dv)
                    for t in range(dv):
                        sl = pl.ds(t * _LANES, _LANES)
                        rows_v[k, sl] = rows_v[k, sl] + accs[t]

            return 0

        lax.fori_loop(0, nch, chunk_body, 0)
        pltpu.sync_copy(rows_v, out_hbm.at[pl.ds(wid * spw, spw)])

    return seg_sum_kernel(x, ed_idx)


def _tc_tail_kernel(sums_ref, inv_counts_ref, clsT_ref, bias_ref, out_ref):
    ave = sums_ref[...] * inv_counts_ref[...]
    t = ave[:, 0:1]
    inner = jnp.sum(ave * ave, axis=1, keepdims=True) - 2.0 * t * t
    denom = jnp.sqrt(jnp.maximum(jnp.abs(inner), 1e-8))
    cx = ave / denom
    col = lax.broadcasted_iota(jnp.int32, cx.shape, 1)
    cx = jnp.where(col == 0, -cx, cx)
    logits = jnp.dot(cx, clsT_ref[...], preferred_element_type=jnp.float32)
    out_ref[...] = 2.0 + 2.0 * logits + bias_ref[...]


def kernel(x, ed_idx, cls, bias):
    b = ed_idx.shape[0]
    c = cls.shape[0]
    sums = _sc_segment_sums(x, ed_idx)
    starts = jnp.concatenate([jnp.zeros((1,), ed_idx.dtype), ed_idx[:-1]])
    counts = jnp.maximum((ed_idx - starts).astype(jnp.float32), 1.0)
    inv_counts = (1.0 / counts)[:, None]
    out = pl.pallas_call(
        _tc_tail_kernel,
        out_shape=jax.ShapeDtypeStruct((b, c), jnp.float32),
    )(sums, inv_counts, cls.T, bias[None, :])
    return out
